# baseline (speedup 1.0000x reference)
import functools

import jax
import jax.numpy as jnp
from jax import lax
from jax.experimental import pallas as pl
from jax.experimental.pallas import tpu as pltpu
from jax.experimental.pallas import tpu_sc as plsc

N = 10000
E = 320000
D = 128

NC = 2
NS = 16
NW = NC * NS
C = 125
ROWS = E // C
CPW = ROWS // NW
NP = 10240
NPT = NP // NS

NB = 2
NITER = CPW // NB

_MESH = plsc.VectorSubcoreMesh(
    core_axis_name="c", subcore_axis_name="s", num_cores=NC, num_subcores=NS
)


def _agg_body(zeros_hbm, h_hbm, src_hbm, dst_hbm, out_hbm,
              src_v, dstr, msg_v, gsem, dsem, agg_sh):
    c = lax.axis_index("c")
    s = lax.axis_index("s")
    w = c * NS + s
    pltpu.sync_copy(
        zeros_hbm.at[pl.ds(s * NPT, NPT)], agg_sh.at[pl.ds(s * NPT, NPT)]
    )
    pltpu.sync_copy(src_hbm.at[pl.ds(w * CPW, CPW)], src_v)
    plsc.subcore_barrier()

    def fire(j, b):
        pltpu.async_copy(h_hbm.at[src_v.at[j]], msg_v.at[b], gsem.at[b])
        pltpu.async_copy(dst_hbm.at[w * CPW + j], dstr.at[b], dsem.at[b])

    for b in range(NB):
        fire(b, b)

    def step(jj, carry):
        base = jj * NB
        for b in range(NB):
            j = base + b
            pltpu.make_async_copy(
                h_hbm.at[src_v.at[j]], msg_v.at[b], gsem.at[b]
            ).wait()
            pltpu.make_async_copy(
                dst_hbm.at[w * CPW + j], dstr.at[b], dsem.at[b]
            ).wait()
            pltpu.sync_copy(msg_v.at[b], agg_sh.at[dstr.at[b, 0]], add=True)

            @pl.when(jj < NITER - 1)
            def _refire():
                fire(j + NB, b)

        return carry

    lax.fori_loop(0, NITER, step, 0)
    plsc.subcore_barrier()
    pltpu.sync_copy(
        agg_sh.at[pl.ds(s * NPT, NPT)], out_hbm.at[c, pl.ds(s * NPT, NPT)]
    )


def _deg_body(zeros_hbm, ones_hbm, dst_hbm, out_hbm,
              dstr, ones_v, dsem, deg_sh):
    c = lax.axis_index("c")
    s = lax.axis_index("s")
    w = c * NS + s
    pltpu.sync_copy(
        zeros_hbm.at[pl.ds(s * NPT, NPT)], deg_sh.at[pl.ds(s * NPT, NPT)]
    )
    pltpu.sync_copy(ones_hbm, ones_v)
    plsc.subcore_barrier()

    for b in range(NB):
        pltpu.async_copy(dst_hbm.at[w * CPW + b], dstr.at[b], dsem.at[b])

    def step(jj, carry):
        base = jj * NB
        for b in range(NB):
            j = base + b
            pltpu.make_async_copy(
                dst_hbm.at[w * CPW + j], dstr.at[b], dsem.at[b]
            ).wait()
            pltpu.sync_copy(ones_v, deg_sh.at[dstr.at[b, 0]], add=True)

            @pl.when(jj < NITER - 1)
            def _refire():
                pltpu.async_copy(
                    dst_hbm.at[w * CPW + j + NB], dstr.at[b], dsem.at[b]
                )

        return carry

    lax.fori_loop(0, NITER, step, 0)
    plsc.subcore_barrier()
    pltpu.sync_copy(
        deg_sh.at[pl.ds(s * NPT, NPT)], out_hbm.at[c, pl.ds(s * NPT, NPT)]
    )


_agg_call = pl.kernel(
    _agg_body,
    out_type=jax.ShapeDtypeStruct((NC, NP, D), jnp.float32),
    mesh=_MESH,
    scratch_types=[
        pltpu.VMEM((CPW, C), jnp.int32),
        pltpu.VMEM((NB, 1, C), jnp.int32),
        pltpu.VMEM((NB, C, D), jnp.float32),
        pltpu.SemaphoreType.DMA((NB,)),
        pltpu.SemaphoreType.DMA((NB,)),
        pltpu.VMEM_SHARED((NP, D), jnp.float32),
    ],
)

_deg_call = pl.kernel(
    _deg_body,
    out_type=jax.ShapeDtypeStruct((NC, NP, D), jnp.float32),
    mesh=_MESH,
    scratch_types=[
        pltpu.VMEM((NB, 1, C), jnp.int32),
        pltpu.VMEM((C, D), jnp.float32),
        pltpu.SemaphoreType.DMA((NB,)),
        pltpu.VMEM_SHARED((NP, D), jnp.float32),
    ],
)


def _rdeg_body(degp_ref, out_ref):
    deg = degp_ref[0, :, :1] + degp_ref[1, :, :1]
    out_ref[...] = 1.0 / jnp.maximum(deg, 1.0)


def _rdeg(degp):
    return pl.pallas_call(
        _rdeg_body,
        grid=(NP // 1024,),
        in_specs=[pl.BlockSpec((NC, 1024, D), lambda i: (0, i, 0))],
        out_specs=pl.BlockSpec((1024, 1), lambda i: (i, 0)),
        out_shape=jax.ShapeDtypeStruct((NP, 1), jnp.float32),
    )(degp)


def _mm_body(h_ref, ws_ref, out_ref):
    out_ref[...] = jnp.dot(
        h_ref[...], ws_ref[...], preferred_element_type=jnp.float32
    )


def _mm(h, ws):
    return pl.pallas_call(
        _mm_body,
        grid=(NP // 1024,),
        in_specs=[
            pl.BlockSpec((1024, D), lambda i: (i, 0)),
            pl.BlockSpec((D, D), lambda i: (0, 0)),
        ],
        out_specs=pl.BlockSpec((1024, D), lambda i: (i, 0)),
        out_shape=jax.ShapeDtypeStruct((NP, D), jnp.float32),
    )(h, ws)


def _combine_body(relu, agg_ref, rdeg_ref, self_ref, wn_ref, out_ref):
    m = (agg_ref[0] + agg_ref[1]) * rdeg_ref[...]
    o = jnp.dot(m, wn_ref[...], preferred_element_type=jnp.float32)
    o += self_ref[...]
    if relu:
        o = jnp.maximum(o, 0.0)
    out_ref[...] = o


def _combine(agg, rdeg, selfp, wn, relu, rows_out, bf):
    grid = (rows_out // bf,)
    return pl.pallas_call(
        functools.partial(_combine_body, relu),
        grid=grid,
        in_specs=[
            pl.BlockSpec((NC, bf, D), lambda i: (0, i, 0)),
            pl.BlockSpec((bf, 1), lambda i: (i, 0)),
            pl.BlockSpec((bf, D), lambda i: (i, 0)),
            pl.BlockSpec((D, D), lambda i: (0, 0)),
        ],
        out_specs=pl.BlockSpec((bf, D), lambda i: (i, 0)),
        out_shape=jax.ShapeDtypeStruct((rows_out, D), jnp.float32),
    )(agg, rdeg, selfp, wn)


def kernel(x, edge_index, Wn0, Ws0, Wn1, Ws1, Wn2, Ws2):
    src2 = edge_index[0].reshape(ROWS, C)
    dst2 = edge_index[1].reshape(ROWS, 1, C)
    zeros_d = jnp.zeros((NP, D), jnp.float32)
    ones_c = jnp.ones((C, D), jnp.float32)

    h = jnp.pad(x, ((0, NP - N), (0, 0)))
    degp = _deg_call(zeros_d, ones_c, dst2)
    selfp = _mm(h, Ws0)
    rdeg = _rdeg(degp)
    for i, (wn, ws) in enumerate(((Wn0, Ws0), (Wn1, Ws1), (Wn2, Ws2))):
        agg = _agg_call(zeros_d, h, src2, dst2)
        last = i == 2
        h = _combine(agg, rdeg, selfp, wn, relu=not last,
                     rows_out=(N if last else NP), bf=(1000 if last else 1024))
        if not last:
            selfp = _mm(h, (Ws1, Ws2)[i])
    return h


# speedup vs baseline: 1.0149x; 1.0149x over previous
import functools

import jax
import jax.numpy as jnp
from jax import lax
from jax.experimental import pallas as pl
from jax.experimental.pallas import tpu as pltpu
from jax.experimental.pallas import tpu_sc as plsc

N = 10000
E = 320000
D = 128

NC = 2
NS = 16
NW = NC * NS
C = 125
ROWS = E // C
CPW = ROWS // NW
NP = 10240
NPT = NP // NS

NB = 2
NITER = CPW // NB

_MESH = plsc.VectorSubcoreMesh(
    core_axis_name="c", subcore_axis_name="s", num_cores=NC, num_subcores=NS
)


def _agg_body(zeros_hbm, h_hbm, e_hbm, out_hbm,
              src_v, dstr, msg_v, gsem, dsem, agg_sh):
    c = lax.axis_index("c")
    s = lax.axis_index("s")
    w = c * NS + s
    pltpu.sync_copy(
        zeros_hbm.at[pl.ds(s * NPT, NPT)], agg_sh.at[pl.ds(s * NPT, NPT)]
    )
    pltpu.sync_copy(e_hbm.at[0, pl.ds(w * CPW, CPW)], src_v)
    plsc.subcore_barrier()

    def fire(j, b):
        pltpu.async_copy(h_hbm.at[src_v.at[j, 0]], msg_v.at[b], gsem.at[b])
        pltpu.async_copy(e_hbm.at[1, w * CPW + j], dstr.at[b], dsem.at[b])

    for b in range(NB):
        fire(b, b)

    def step(jj, carry):
        base = jj * NB
        for b in range(NB):
            j = base + b
            pltpu.make_async_copy(
                h_hbm.at[src_v.at[j, 0]], msg_v.at[b], gsem.at[b]
            ).wait()
            pltpu.make_async_copy(
                e_hbm.at[1, w * CPW + j], dstr.at[b], dsem.at[b]
            ).wait()
            pltpu.sync_copy(msg_v.at[b], agg_sh.at[dstr.at[b, 0]], add=True)

            @pl.when(jj < NITER - 1)
            def _refire():
                fire(j + NB, b)

        return carry

    lax.fori_loop(0, NITER, step, 0)
    plsc.subcore_barrier()
    pltpu.sync_copy(
        agg_sh.at[pl.ds(s * NPT, NPT)], out_hbm.at[c, pl.ds(s * NPT, NPT)]
    )


def _deg_body(zeros_hbm, ones_hbm, e_hbm, out_hbm,
              dstr, ones_v, dsem, deg_sh):
    c = lax.axis_index("c")
    s = lax.axis_index("s")
    w = c * NS + s
    pltpu.sync_copy(
        zeros_hbm.at[pl.ds(s * NPT, NPT)], deg_sh.at[pl.ds(s * NPT, NPT)]
    )
    pltpu.sync_copy(ones_hbm, ones_v)
    plsc.subcore_barrier()

    for b in range(NB):
        pltpu.async_copy(e_hbm.at[1, w * CPW + b], dstr.at[b], dsem.at[b])

    def step(jj, carry):
        base = jj * NB
        for b in range(NB):
            j = base + b
            pltpu.make_async_copy(
                e_hbm.at[1, w * CPW + j], dstr.at[b], dsem.at[b]
            ).wait()
            pltpu.sync_copy(ones_v, deg_sh.at[dstr.at[b, 0]], add=True)

            @pl.when(jj < NITER - 1)
            def _refire():
                pltpu.async_copy(
                    e_hbm.at[1, w * CPW + j + NB], dstr.at[b], dsem.at[b]
                )

        return carry

    lax.fori_loop(0, NITER, step, 0)
    plsc.subcore_barrier()
    pltpu.sync_copy(
        deg_sh.at[pl.ds(s * NPT, NPT)], out_hbm.at[c, pl.ds(s * NPT, NPT)]
    )


_agg_call = pl.kernel(
    _agg_body,
    out_type=jax.ShapeDtypeStruct((NC, NP, D), jnp.float32),
    mesh=_MESH,
    scratch_types=[
        pltpu.VMEM((CPW, 1, C), jnp.int32),
        pltpu.VMEM((NB, 1, C), jnp.int32),
        pltpu.VMEM((NB, C, D), jnp.float32),
        pltpu.SemaphoreType.DMA((NB,)),
        pltpu.SemaphoreType.DMA((NB,)),
        pltpu.VMEM_SHARED((NP, D), jnp.float32),
    ],
)

_deg_call = pl.kernel(
    _deg_body,
    out_type=jax.ShapeDtypeStruct((NC, NP, D), jnp.float32),
    mesh=_MESH,
    scratch_types=[
        pltpu.VMEM((NB, 1, C), jnp.int32),
        pltpu.VMEM((C, D), jnp.float32),
        pltpu.SemaphoreType.DMA((NB,)),
        pltpu.VMEM_SHARED((NP, D), jnp.float32),
    ],
)

_BF = 1000


def _rdeg_body(degp_ref, out_ref):
    deg = degp_ref[0, :, :1] + degp_ref[1, :, :1]
    out_ref[...] = 1.0 / jnp.maximum(deg, 1.0)


def _rdeg(degp):
    return pl.pallas_call(
        _rdeg_body,
        grid=(N // _BF,),
        in_specs=[pl.BlockSpec((NC, _BF, D), lambda i: (0, i, 0))],
        out_specs=pl.BlockSpec((_BF, 1), lambda i: (i, 0)),
        out_shape=jax.ShapeDtypeStruct((N, 1), jnp.float32),
    )(degp)


def _mm_body(h_ref, ws_ref, out_ref):
    out_ref[...] = jnp.dot(
        h_ref[...], ws_ref[...], preferred_element_type=jnp.float32
    )


def _mm(h, ws):
    return pl.pallas_call(
        _mm_body,
        grid=(N // _BF,),
        in_specs=[
            pl.BlockSpec((_BF, D), lambda i: (i, 0)),
            pl.BlockSpec((D, D), lambda i: (0, 0)),
        ],
        out_specs=pl.BlockSpec((_BF, D), lambda i: (i, 0)),
        out_shape=jax.ShapeDtypeStruct((N, D), jnp.float32),
    )(h, ws)


def _combine_body(relu, agg_ref, rdeg_ref, self_ref, wn_ref, out_ref):
    m = (agg_ref[0] + agg_ref[1]) * rdeg_ref[...]
    o = jnp.dot(m, wn_ref[...], preferred_element_type=jnp.float32)
    o += self_ref[...]
    if relu:
        o = jnp.maximum(o, 0.0)
    out_ref[...] = o


def _combine(agg, rdeg, selfp, wn, relu):
    return pl.pallas_call(
        functools.partial(_combine_body, relu),
        grid=(N // _BF,),
        in_specs=[
            pl.BlockSpec((NC, _BF, D), lambda i: (0, i, 0)),
            pl.BlockSpec((_BF, 1), lambda i: (i, 0)),
            pl.BlockSpec((_BF, D), lambda i: (i, 0)),
            pl.BlockSpec((D, D), lambda i: (0, 0)),
        ],
        out_specs=pl.BlockSpec((_BF, D), lambda i: (i, 0)),
        out_shape=jax.ShapeDtypeStruct((N, D), jnp.float32),
    )(agg, rdeg, selfp, wn)


def kernel(x, edge_index, Wn0, Ws0, Wn1, Ws1, Wn2, Ws2):
    e4 = edge_index.reshape(2, ROWS, 1, C)
    zeros_d = jnp.zeros((NP, D), jnp.float32)
    ones_c = jnp.ones((C, D), jnp.float32)

    degp = _deg_call(zeros_d, ones_c, e4)
    selfp = _mm(x, Ws0)
    rdeg = _rdeg(degp)
    h = x
    for i, (wn, ws) in enumerate(((Wn0, Ws0), (Wn1, Ws1), (Wn2, Ws2))):
        agg = _agg_call(zeros_d, h, e4)
        h = _combine(agg, rdeg, selfp, wn, relu=(i < 2))
        if not last_layer(i):
            selfp = _mm(h, (Ws1, Ws2)[i])
    return h


def last_layer(i):
    return i == 2


# speedup vs baseline: 1.0363x; 1.0210x over previous
import functools

import jax
import jax.numpy as jnp
from jax import lax
from jax.experimental import pallas as pl
from jax.experimental.pallas import tpu as pltpu
from jax.experimental.pallas import tpu_sc as plsc

N = 10000
E = 320000
D = 128

NC = 2
NS = 16
NW = NC * NS
C = 125
ROWS = E // C
CPW = ROWS // NW
NP = 10240
NPT = NP // NS

NB = 2
NITER = CPW // NB

_MESH = plsc.VectorSubcoreMesh(
    core_axis_name="c", subcore_axis_name="s", num_cores=NC, num_subcores=NS
)


def _agg_body(zeros_hbm, h_hbm, e_hbm, out_hbm,
              src_v, dstr, msg_v, gsem, dsem, agg_sh):
    c = lax.axis_index("c")
    s = lax.axis_index("s")
    w = c * NS + s
    pltpu.sync_copy(e_hbm.at[0, pl.ds(w * CPW, CPW)], src_v)

    def fire(j, b):
        pltpu.async_copy(h_hbm.at[src_v.at[j, 0]], msg_v.at[b], gsem.at[b])
        pltpu.async_copy(e_hbm.at[1, w * CPW + j], dstr.at[b], dsem.at[b])

    for b in range(NB):
        fire(b, b)

    pltpu.sync_copy(
        zeros_hbm.at[pl.ds(s * NPT, NPT)], agg_sh.at[pl.ds(s * NPT, NPT)]
    )
    plsc.subcore_barrier()

    def step(jj, carry):
        base = jj * NB
        for b in range(NB):
            j = base + b
            pltpu.make_async_copy(
                h_hbm.at[src_v.at[j, 0]], msg_v.at[b], gsem.at[b]
            ).wait()
            pltpu.make_async_copy(
                e_hbm.at[1, w * CPW + j], dstr.at[b], dsem.at[b]
            ).wait()
            pltpu.sync_copy(msg_v.at[b], agg_sh.at[dstr.at[b, 0]], add=True)

            @pl.when(jj < NITER - 1)
            def _refire():
                fire(j + NB, b)

        return carry

    lax.fori_loop(0, NITER, step, 0)
    plsc.subcore_barrier()
    pltpu.sync_copy(
        agg_sh.at[pl.ds(s * NPT, NPT)], out_hbm.at[c, pl.ds(s * NPT, NPT)]
    )


def _deg_body(zeros_hbm, ones_hbm, e_hbm, out_hbm,
              dstr, ones_v, dsem, deg_sh):
    c = lax.axis_index("c")
    s = lax.axis_index("s")
    w = c * NS + s
    pltpu.sync_copy(ones_hbm, ones_v)

    for b in range(NB):
        pltpu.async_copy(e_hbm.at[1, w * CPW + b], dstr.at[b], dsem.at[b])

    pltpu.sync_copy(
        zeros_hbm.at[pl.ds(s * NPT, NPT)], deg_sh.at[pl.ds(s * NPT, NPT)]
    )
    plsc.subcore_barrier()

    def step(jj, carry):
        base = jj * NB
        for b in range(NB):
            j = base + b
            pltpu.make_async_copy(
                e_hbm.at[1, w * CPW + j], dstr.at[b], dsem.at[b]
            ).wait()
            pltpu.sync_copy(ones_v, deg_sh.at[dstr.at[b, 0]], add=True)

            @pl.when(jj < NITER - 1)
            def _refire():
                pltpu.async_copy(
                    e_hbm.at[1, w * CPW + j + NB], dstr.at[b], dsem.at[b]
                )

        return carry

    lax.fori_loop(0, NITER, step, 0)
    plsc.subcore_barrier()
    pltpu.sync_copy(
        deg_sh.at[pl.ds(s * NPT, NPT)], out_hbm.at[c, pl.ds(s * NPT, NPT)]
    )


_agg_call = pl.kernel(
    _agg_body,
    out_type=jax.ShapeDtypeStruct((NC, NP, D), jnp.float32),
    mesh=_MESH,
    scratch_types=[
        pltpu.VMEM((CPW, 1, C), jnp.int32),
        pltpu.VMEM((NB, 1, C), jnp.int32),
        pltpu.VMEM((NB, C, D), jnp.float32),
        pltpu.SemaphoreType.DMA((NB,)),
        pltpu.SemaphoreType.DMA((NB,)),
        pltpu.VMEM_SHARED((NP, D), jnp.float32),
    ],
)

_deg_call = pl.kernel(
    _deg_body,
    out_type=jax.ShapeDtypeStruct((NC, NP, D), jnp.float32),
    mesh=_MESH,
    scratch_types=[
        pltpu.VMEM((NB, 1, C), jnp.int32),
        pltpu.VMEM((C, D), jnp.float32),
        pltpu.SemaphoreType.DMA((NB,)),
        pltpu.VMEM_SHARED((NP, D), jnp.float32),
    ],
)

_BF = 2000


def _rdeg_body(degp_ref, out_ref):
    deg = degp_ref[0, :, :1] + degp_ref[1, :, :1]
    out_ref[...] = 1.0 / jnp.maximum(deg, 1.0)


def _rdeg(degp):
    return pl.pallas_call(
        _rdeg_body,
        grid=(N // _BF,),
        in_specs=[pl.BlockSpec((NC, _BF, D), lambda i: (0, i, 0))],
        out_specs=pl.BlockSpec((_BF, 1), lambda i: (i, 0)),
        out_shape=jax.ShapeDtypeStruct((N, 1), jnp.float32),
    )(degp)


def _mm_body(h_ref, ws_ref, out_ref):
    out_ref[...] = jnp.dot(
        h_ref[...], ws_ref[...], preferred_element_type=jnp.float32
    )


def _mm(h, ws):
    return pl.pallas_call(
        _mm_body,
        grid=(N // _BF,),
        in_specs=[
            pl.BlockSpec((_BF, D), lambda i: (i, 0)),
            pl.BlockSpec((D, D), lambda i: (0, 0)),
        ],
        out_specs=pl.BlockSpec((_BF, D), lambda i: (i, 0)),
        out_shape=jax.ShapeDtypeStruct((N, D), jnp.float32),
    )(h, ws)


def _combine_body(relu, agg_ref, rdeg_ref, self_ref, wn_ref, out_ref):
    m = (agg_ref[0] + agg_ref[1]) * rdeg_ref[...]
    o = jnp.dot(m, wn_ref[...], preferred_element_type=jnp.float32)
    o += self_ref[...]
    if relu:
        o = jnp.maximum(o, 0.0)
    out_ref[...] = o


def _combine(agg, rdeg, selfp, wn, relu):
    return pl.pallas_call(
        functools.partial(_combine_body, relu),
        grid=(N // _BF,),
        in_specs=[
            pl.BlockSpec((NC, _BF, D), lambda i: (0, i, 0)),
            pl.BlockSpec((_BF, 1), lambda i: (i, 0)),
            pl.BlockSpec((_BF, D), lambda i: (i, 0)),
            pl.BlockSpec((D, D), lambda i: (0, 0)),
        ],
        out_specs=pl.BlockSpec((_BF, D), lambda i: (i, 0)),
        out_shape=jax.ShapeDtypeStruct((N, D), jnp.float32),
    )(agg, rdeg, selfp, wn)


def kernel(x, edge_index, Wn0, Ws0, Wn1, Ws1, Wn2, Ws2):
    e4 = edge_index.reshape(2, ROWS, 1, C)
    zeros_d = jnp.zeros((NP, D), jnp.float32)
    ones_c = jnp.ones((C, D), jnp.float32)

    degp = _deg_call(zeros_d, ones_c, e4)
    selfp = _mm(x, Ws0)
    rdeg = _rdeg(degp)
    h = x
    for i, (wn, ws) in enumerate(((Wn0, Ws0), (Wn1, Ws1), (Wn2, Ws2))):
        agg = _agg_call(zeros_d, h, e4)
        h = _combine(agg, rdeg, selfp, wn, relu=(i < 2))
        if not last_layer(i):
            selfp = _mm(h, (Ws1, Ws2)[i])
    return h


def last_layer(i):
    return i == 2
